# baseline (speedup 1.0000x reference)
import functools

import jax
import jax.numpy as jnp
from jax import lax
from jax.experimental import pallas as pl
from jax.experimental.pallas import tpu as pltpu
from jax.experimental.pallas import tpu_sc as plsc

VOCAB = 99996
DIM = 64
BATCH = 4096
FIELDS = 26

NC = 2
NS = 16
NW = NC * NS
B_PER_W = BATCH // NW
ROWS_PER_CHUNK = 16
NCHUNK = B_PER_W // ROWS_PER_CHUNK
SUB = 4
IDX_PER_SUB = ROWS_PER_CHUNK * FIELDS // SUB
NVREG = DIM // 16


def _fm_body(x_hbm, table_hbm, out_hbm, idx_v, rows_v, out_v, sem0, sem1):
    wid = lax.axis_index("s") * NC + lax.axis_index("c")
    pltpu.sync_copy(x_hbm.at[wid], idx_v)
    lane = lax.broadcasted_iota(jnp.int32, (16,), 0)
    perms = [lane ^ sh for sh in (8, 4, 2, 1)]
    sems = (sem0, sem1)

    def fire(c, buf):
        for s in range(SUB):
            pltpu.async_copy(
                table_hbm.at[idx_v.at[c, s]],
                rows_v.at[buf, pl.ds(s * IDX_PER_SUB, IDX_PER_SUB)],
                sems[buf],
            )

    def drain(buf):
        pltpu.make_async_copy(
            table_hbm.at[pl.ds(0, ROWS_PER_CHUNK * FIELDS)],
            rows_v.at[buf],
            sems[buf],
        ).wait()

    def compute(c, buf):
        out_vec = jnp.zeros((16,), jnp.float32)
        for j in range(ROWS_PER_CHUNK):
            acc = [jnp.zeros((16,), jnp.float32) for _ in range(NVREG)]
            accq = jnp.zeros((16,), jnp.float32)
            for f in range(FIELDS):
                r = j * FIELDS + f
                for i in range(NVREG):
                    v = rows_v[buf, r, pl.ds(i * 16, 16)]
                    acc[i] = acc[i] + v
                    accq = accq + v * v
            tot = -accq
            for i in range(NVREG):
                tot = tot + acc[i] * acc[i]
            for p in perms:
                tot = tot + tot.at[p].get(mode="promise_in_bounds")
            out_vec = jnp.where(lane == j, tot, out_vec)
        out_v[c] = out_vec

    NGROUP = NCHUNK // 2
    fire(0, 0)

    def group_body(g, carry):
        c0 = 2 * g
        fire(c0 + 1, 1)
        drain(0)
        compute(c0, 0)

        @pl.when(g < NGROUP - 1)
        def _():
            fire(c0 + 2, 0)

        drain(1)
        compute(c0 + 1, 1)
        return carry

    lax.fori_loop(0, NGROUP, group_body, 0)
    pltpu.sync_copy(out_v, out_hbm.at[wid])


@jax.jit
def kernel(x, table):
    xr = x.astype(jnp.int32).reshape(NW, NCHUNK, SUB, IDX_PER_SUB)
    mesh = plsc.VectorSubcoreMesh(core_axis_name="c", subcore_axis_name="s")
    fm = pl.kernel(
        _fm_body,
        out_type=jax.ShapeDtypeStruct((NW, NCHUNK, 16), jnp.float32),
        mesh=mesh,
        scratch_types=[
            pltpu.VMEM((NCHUNK, SUB, IDX_PER_SUB), jnp.int32),
            pltpu.VMEM((2, ROWS_PER_CHUNK * FIELDS, DIM), jnp.float32),
            pltpu.VMEM((NCHUNK, 16), jnp.float32),
            pltpu.SemaphoreType.DMA,
            pltpu.SemaphoreType.DMA,
        ],
        compiler_params=pltpu.CompilerParams(use_tc_tiling_on_sc=False),
    )
    out = fm(xr, table)
    return out.reshape(BATCH)


# speedup vs baseline: 1.2236x; 1.2236x over previous
import functools

import jax
import jax.numpy as jnp
from jax import lax
from jax.experimental import pallas as pl
from jax.experimental.pallas import tpu as pltpu
from jax.experimental.pallas import tpu_sc as plsc

VOCAB = 99996
DIM = 64
BATCH = 4096
FIELDS = 26

NC = 2
NS = 16
NW = NC * NS
B_PER_W = BATCH // NW
ROWS_PER_CHUNK = 16
NCHUNK = B_PER_W // ROWS_PER_CHUNK
SUB = 4
IDX_PER_SUB = ROWS_PER_CHUNK * FIELDS // SUB
NVREG = DIM // 16
NBUF = 4
NGRP = NCHUNK // NBUF


def _fm_body(x_hbm, table_hbm, out_hbm, idx_v, rows_v, out_v,
             sem0, sem1, sem2, sem3):
    wid = lax.axis_index("s") * NC + lax.axis_index("c")
    pltpu.sync_copy(x_hbm.at[wid], idx_v)
    lane = lax.broadcasted_iota(jnp.int32, (16,), 0)
    perms = [lane ^ sh for sh in (8, 4, 2, 1)]
    sems = (sem0, sem1, sem2, sem3)

    def fire(c, buf):
        for s in range(SUB):
            pltpu.async_copy(
                table_hbm.at[idx_v.at[c, s]],
                rows_v.at[buf, pl.ds(s * IDX_PER_SUB, IDX_PER_SUB)],
                sems[buf],
            )

    def drain(buf):
        pltpu.make_async_copy(
            table_hbm.at[pl.ds(0, ROWS_PER_CHUNK * FIELDS)],
            rows_v.at[buf],
            sems[buf],
        ).wait()

    def compute(c, buf):
        def jbody(j, out_vec):
            base = j * FIELDS
            acc = [jnp.zeros((16,), jnp.float32) for _ in range(NVREG)]
            accq = jnp.zeros((16,), jnp.float32)
            for f in range(FIELDS):
                for i in range(NVREG):
                    v = rows_v[buf, base + f, pl.ds(i * 16, 16)]
                    acc[i] = acc[i] + v
                    accq = accq + v * v
            tot = -accq
            for i in range(NVREG):
                tot = tot + acc[i] * acc[i]
            for p in perms:
                tot = tot + tot.at[p].get(mode="promise_in_bounds")
            return jnp.where(lane == j, tot, out_vec)

        out_v[c] = lax.fori_loop(0, ROWS_PER_CHUNK, jbody,
                                 jnp.zeros((16,), jnp.float32))

    for b in range(NBUF - 1):
        fire(b, b)

    def group_body(g, carry):
        for b in range(NBUF):
            c = NBUF * g + b
            drain(b)
            compute(c, b)
            nxt = c + NBUF - 1

            @pl.when(nxt < NCHUNK)
            def _():
                fire(nxt, (b + NBUF - 1) % NBUF)

        return carry

    lax.fori_loop(0, NGRP, group_body, 0)
    pltpu.sync_copy(out_v, out_hbm.at[wid])


@jax.jit
def kernel(x, table):
    xr = x.astype(jnp.int32).reshape(NW, NCHUNK, SUB, IDX_PER_SUB)
    mesh = plsc.VectorSubcoreMesh(core_axis_name="c", subcore_axis_name="s")
    fm = pl.kernel(
        _fm_body,
        out_type=jax.ShapeDtypeStruct((NW, NCHUNK, 16), jnp.float32),
        mesh=mesh,
        scratch_types=[
            pltpu.VMEM((NCHUNK, SUB, IDX_PER_SUB), jnp.int32),
            pltpu.VMEM((NBUF, ROWS_PER_CHUNK * FIELDS, DIM), jnp.float32),
            pltpu.VMEM((NCHUNK, 16), jnp.float32),
            pltpu.SemaphoreType.DMA,
            pltpu.SemaphoreType.DMA,
            pltpu.SemaphoreType.DMA,
            pltpu.SemaphoreType.DMA,
        ],
        compiler_params=pltpu.CompilerParams(use_tc_tiling_on_sc=False),
    )
    out = fm(xr, table)
    return out.reshape(BATCH)
